# baseline (speedup 1.0000x reference)
import functools

import jax
import jax.numpy as jnp
from jax import lax
from jax.experimental import pallas as pl
from jax.experimental.pallas import tpu as pltpu
from jax.experimental.pallas import tpu_sc as plsc

N_EDGES = 1600000
NUM_WORKERS = 32
EDGES_PER_WORKER = N_EDGES // NUM_WORKERS
CHUNK = 400
NCHUNKS = EDGES_PER_WORKER // CHUNK
IDX_MINOR = 80
IDX_ROWS = CHUNK // IDX_MINOR
GROUPS = CHUNK // 16


def _col_of(m, i):
    return m if i == 0 else 3 * m + 3 + i


def _outcol(n, o):
    return n if o == 0 else 3 * n + 3 + o


def _make_sc_kernel():
    mesh = plsc.VectorSubcoreMesh(core_axis_name="c", subcore_axis_name="s")

    vm = pltpu.VMEM
    scratch = (
        [vm((CHUNK,), jnp.int32) for _ in range(2)]
        + [vm((CHUNK, 16), jnp.float32) for _ in range(2)]
        + [vm((CHUNK, 4, 4), jnp.float32) for _ in range(4)]
        + [vm((CHUNK, 16), jnp.float32) for _ in range(2)]
        + [pltpu.SemaphoreType.DMA for _ in range(10)]
    )

    @functools.partial(
        pl.kernel,
        mesh=mesh,
        compiler_params=pltpu.CompilerParams(
            needs_layout_passes=False, use_tc_tiling_on_sc=False
        ),
        out_type=jax.ShapeDtypeStruct((N_EDGES, 16), jnp.float32),
        scratch_types=scratch,
    )
    def sc_kernel(u_hbm, b_hbm, w_hbm, nf_hbm, out_hbm,
                  idx0, idx1, f0, f1, b0, b1, w0, w1, o0, o1,
                  si0, si1, sg0, sg1, sb0, sb1, sw0, sw1, so0, so1):
        wid = lax.axis_index("s") * 2 + lax.axis_index("c")
        iota16 = lax.iota(jnp.int32, 16)
        idx_v = (idx0, idx1)
        f_v = (f0, f1)
        b_v = (b0, b1)
        w_v = (w0, w1)
        o_v = (o0, o1)
        sem_i = (si0, si1)
        sem_g = (sg0, sg1)
        sem_b = (sb0, sb1)
        sem_w = (sw0, sw1)
        sem_o = (so0, so1)

        def ebase(c):
            return wid * EDGES_PER_WORKER + c * CHUNK

        def idx_desc(c, p):
            return pltpu.make_async_copy(
                u_hbm.at[0, pl.ds(ebase(c), CHUNK)], idx_v[p], sem_i[p]
            )

        def gather_descs(c, p):
            return [
                pltpu.make_async_copy(
                    nf_hbm.at[idx_v[p].at[pl.ds(j * IDX_MINOR, IDX_MINOR)]],
                    f_v[p].at[pl.ds(j * IDX_MINOR, IDX_MINOR)],
                    sem_g[p],
                )
                for j in range(IDX_ROWS)
            ]

        def b_desc(c, p):
            return pltpu.make_async_copy(
                b_hbm.at[pl.ds(ebase(c), CHUNK)], b_v[p], sem_b[p]
            )

        def w_desc(c, p):
            return pltpu.make_async_copy(
                w_hbm.at[pl.ds(ebase(c), CHUNK)], w_v[p], sem_w[p]
            )

        def out_desc(c, p):
            return pltpu.make_async_copy(
                o_v[p], out_hbm.at[pl.ds(ebase(c), CHUNK)], sem_o[p]
            )

        def issue_inputs(c, p):
            for d in gather_descs(c, p):
                d.start()
            b_desc(c, p).start()
            w_desc(c, p).start()

        def wait_inputs(c, p):
            for d in gather_descs(c, p):
                d.wait()
            b_desc(c, p).wait()
            w_desc(c, p).wait()

        def compute(c, p):
            fp, bp, wp, op = f_v[p], b_v[p], w_v[p], o_v[p]

            def group(g, gcarry):
                eidx = g * 16 + iota16

                def splat(k):
                    return jnp.full((16,), k, jnp.int32)

                def fcol(k):
                    return plsc.load_gather(fp, [eidx, splat(k)])

                def bcol(i, o):
                    return plsc.load_gather(bp, [eidx, splat(i), splat(o)])

                def wcol(n, m):
                    return plsc.load_gather(wp, [eidx, splat(n), splat(m)])

                fc = [fcol(k) for k in range(16)]
                bc = [[bcol(i, o) for o in range(4)] for i in range(4)]
                tmp = []
                for m in range(4):
                    row = []
                    for o in range(4):
                        acc = fc[_col_of(m, 0)] * bc[0][o]
                        for i in range(1, 4):
                            acc = acc + fc[_col_of(m, i)] * bc[i][o]
                        row.append(acc)
                    tmp.append(row)
                wc = [[wcol(n, m) for m in range(4)] for n in range(4)]
                for n in range(4):
                    for o in range(4):
                        acc = wc[n][0] * tmp[0][o]
                        for m in range(1, 4):
                            acc = acc + wc[n][m] * tmp[m][o]
                        plsc.store_scatter(op, [eidx, splat(_outcol(n, o))], acc)
                return gcarry

            lax.fori_loop(0, GROUPS, group, 0)

        def process(c, p):
            q = 1 - p
            @pl.when(c + 1 < NCHUNKS)
            def _():
                idx_desc(c + 1, q).wait()
                issue_inputs(c + 1, q)

            wait_inputs(c, p)

            @pl.when(c + 2 < NCHUNKS)
            def _():
                idx_desc(c + 2, p).start()

            @pl.when(c >= 2)
            def _():
                out_desc(c - 2, p).wait()

            compute(c, p)
            out_desc(c, p).start()

        idx_desc(0, 0).start()
        idx_desc(0, 0).wait()
        issue_inputs(0, 0)
        idx_desc(1, 1).start()

        @pl.loop(0, NCHUNKS - 1, step=2)
        def _(c):
            process(c, 0)
            process(c + 1, 1)

        process(jnp.int32(NCHUNKS - 1), 0)

        out_desc(NCHUNKS - 2, 1).wait()
        out_desc(NCHUNKS - 1, 0).wait()

    return sc_kernel


_SC_KERNEL = _make_sc_kernel()


def kernel(edge_index, basis, edge_weights, node_features):
    return _SC_KERNEL(edge_index, basis, edge_weights, node_features)


# speedup vs baseline: 20.0674x; 20.0674x over previous
import functools

import jax
import jax.numpy as jnp
from jax import lax
from jax.experimental import pallas as pl
from jax.experimental.pallas import tpu as pltpu
from jax.experimental.pallas import tpu_sc as plsc

N_EDGES = 1600000
LANES = 128
KB = N_EDGES // LANES
NB = 4
CHUNK = NB * LANES
TCH = KB // NB
NUM_WORKERS = 32
CPW = -(-TCH // NUM_WORKERS)
GROUPS = CHUNK // 16


def _col_of(m, i):
    return m if i == 0 else 3 * m + 3 + i


def _outcol(n, o):
    return n if o == 0 else 3 * n + 3 + o


def _make_sc_kernel():
    mesh = plsc.VectorSubcoreMesh(core_axis_name="c", subcore_axis_name="s")

    vm = pltpu.VMEM
    scratch = (
        [vm((NB, 2, LANES), jnp.int32) for _ in range(2)]
        + [vm((CHUNK, 16), jnp.float32) for _ in range(2)]
        + [vm((4, NB, 4, LANES), jnp.float32) for _ in range(4)]
        + [vm((CHUNK, 16), jnp.float32) for _ in range(2)]
        + [pltpu.SemaphoreType.DMA for _ in range(10)]
    )

    @functools.partial(
        pl.kernel,
        mesh=mesh,
        compiler_params=pltpu.CompilerParams(
            needs_layout_passes=False, use_tc_tiling_on_sc=False
        ),
        out_type=jax.ShapeDtypeStruct((N_EDGES, 16), jnp.float32),
        scratch_types=scratch,
    )
    def sc_kernel(ei_hbm, b_hbm, w_hbm, nf_hbm, out_hbm,
                  u0, u1, f0, f1, b0, b1, w0, w1, o0, o1,
                  su0, su1, sg0, sg1, sb0, sb1, sw0, sw1, so0, so1):
        wid = lax.axis_index("s") * 2 + lax.axis_index("c")
        start = (wid * TCH) // NUM_WORKERS
        iota16 = lax.iota(jnp.int32, 16)
        u_v = (u0, u1)
        f_v = (f0, f1)
        b_v = (b0, b1)
        w_v = (w0, w1)
        o_v = (o0, o1)
        sem_u = (su0, su1)
        sem_g = (sg0, sg1)
        sem_b = (sb0, sb1)
        sem_w = (sw0, sw1)
        sem_o = (so0, so1)

        def u_desc(t, p):
            kb0 = (start + t) * NB
            return pltpu.make_async_copy(
                ei_hbm.at[pl.ds(kb0, NB)], u_v[p], sem_u[p]
            )

        def gather_descs(t, p):
            return [
                pltpu.make_async_copy(
                    nf_hbm.at[u_v[p].at[j, 0]],
                    f_v[p].at[pl.ds(j * LANES, LANES)],
                    sem_g[p],
                )
                for j in range(NB)
            ]

        def bw_descs(t, p, hbm, buf, sem):
            kb0 = (start + t) * NB
            return [
                pltpu.make_async_copy(
                    hbm.at[i, pl.ds(kb0, NB)], buf[p].at[i], sem[p]
                )
                for i in range(4)
            ]

        def out_desc(t, p):
            e0 = (start + t) * CHUNK
            return pltpu.make_async_copy(
                o_v[p], out_hbm.at[pl.ds(e0, CHUNK)], sem_o[p]
            )

        def issue_inputs(t, p):
            for d in gather_descs(t, p):
                d.start()
            for d in bw_descs(t, p, b_hbm, b_v, sem_b):
                d.start()
            for d in bw_descs(t, p, w_hbm, w_v, sem_w):
                d.start()

        def wait_inputs(t, p):
            for d in gather_descs(t, p):
                d.wait()
            for d in bw_descs(t, p, b_hbm, b_v, sem_b):
                d.wait()
            for d in bw_descs(t, p, w_hbm, w_v, sem_w):
                d.wait()

        def compute(t, p):
            fp, bp, wp, op = f_v[p], b_v[p], w_v[p], o_v[p]

            def group(g, gcarry):
                kb = lax.shift_right_logical(g, 1)
                half = lax.bitwise_and(g, 1)
                lb = half * 64

                def do_sub(s):
                    lo = lb + s * 16
                    eloc = kb * LANES + lo + iota16

                    def fcol(k):
                        kvec = jnp.full((16,), k, jnp.int32)
                        return plsc.load_gather(fp, [eloc, kvec])

                    fc = [fcol(k) for k in range(16)]
                    bc = [[bp[i, kb, o, pl.ds(lo, 16)] for o in range(4)]
                          for i in range(4)]
                    tmp = []
                    for m in range(4):
                        row = []
                        for o in range(4):
                            acc = fc[_col_of(m, 0)] * bc[0][o]
                            for i in range(1, 4):
                                acc = acc + fc[_col_of(m, i)] * bc[i][o]
                            row.append(acc)
                        tmp.append(row)
                    wc = [[wp[n, kb, m, pl.ds(lo, 16)] for m in range(4)]
                          for n in range(4)]
                    for n in range(4):
                        for o in range(4):
                            acc = wc[n][0] * tmp[0][o]
                            for m in range(1, 4):
                                acc = acc + wc[n][m] * tmp[m][o]
                            kvec = jnp.full((16,), _outcol(n, o), jnp.int32)
                            plsc.store_scatter(op, [eloc, kvec], acc)

                for s in range(4):
                    do_sub(s)
                return gcarry

            lax.fori_loop(0, GROUPS // 4, group, 0)

        def process(t, p):
            q = 1 - p
            @pl.when(t + 1 < CPW)
            def _():
                u_desc(t + 1, q).wait()
                issue_inputs(t + 1, q)

            wait_inputs(t, p)

            @pl.when(t + 2 < CPW)
            def _():
                u_desc(t + 2, p).start()

            @pl.when(t >= 2)
            def _():
                out_desc(t - 2, p).wait()

            compute(t, p)
            out_desc(t, p).start()

        u_desc(0, 0).start()
        u_desc(0, 0).wait()
        issue_inputs(0, 0)
        u_desc(1, 1).start()

        @pl.loop(0, CPW - 1, step=2)
        def _(t):
            process(t, 0)
            process(t + 1, 1)

        out_desc(CPW - 2, 0).wait()
        out_desc(CPW - 1, 1).wait()

    return sc_kernel


_SC_KERNEL = _make_sc_kernel()


def kernel(edge_index, basis, edge_weights, node_features):
    e = basis.shape[0]
    kb = e // LANES
    b_x = basis.reshape(kb, LANES, 4, 4).transpose(2, 0, 3, 1)
    w_x = edge_weights.reshape(kb, LANES, 4, 4).transpose(2, 0, 3, 1)
    ei_x = edge_index.reshape(2, kb, LANES).transpose(1, 0, 2)
    return _SC_KERNEL(ei_x, b_x, w_x, node_features)
